# Initial kernel scaffold; baseline (speedup 1.0000x reference)
import jax
import jax.numpy as jnp
from jax.experimental import pallas as pl


def kernel(x, edge_index, W1, b1, W2, b2):
    raise NotImplementedError("write your pallas kernel here")


# speedup vs baseline: 7.7282x; 7.7282x over previous
import jax
import jax.numpy as jnp
from jax import lax
from jax.experimental import pallas as pl
from jax.experimental.pallas import tpu as pltpu
from jax.experimental.pallas import tpu_sc as plsc

N = 10000
E = 320000
IN = 128
HID = 400
OUT = 4
K = 4

NPAD = 10240
NC, NS, L = 2, 16, 16
HALFP = NPAD // NC
PT = HALFP // NS
ACC_ROWS = 5184
ZPT = ACC_ROWS // NS
TRASH = HALFP
EPT = E // NS
NBUF = 5
EPC = 160
SUBS = 2
NCH = EPT // EPC

_MESH = dict(core_axis_name="c", subcore_axis_name="s")
_SC_PARAMS = pltpu.CompilerParams(needs_layout_passes=False,
                                 use_tc_tiling_on_sc=False)


def _fast_rsqrt(d):
    i = lax.bitcast_convert_type(d, jnp.int32)
    i = jnp.int32(0x5F3759DF) - lax.shift_right_arithmetic(i, 1)
    y = lax.bitcast_convert_type(i, jnp.float32)
    for _ in range(3):
        y = y * (1.5 - 0.5 * d * y * y)
    return y


def _zero_acc(zb, acc_sh, s, c):
    def zrow(r, _):
        for j in range(c // L):
            zb[r, pl.ds(j * L, L)] = jnp.zeros((L,), jnp.float32)
        return _
    lax.fori_loop(0, ZPT, zrow, None)
    pltpu.sync_copy(zb, acc_sh.at[pl.ds(s * ZPT, ZPT)])


def _remap_store(ctmp, cidx, b, nodebase):
    for k in range(EPC // L):
        v = ctmp[b][pl.ds(k * L, L)]
        lc = v - nodebase
        ok = (lc >= 0) & (lc < HALFP)
        lc = jnp.where(ok, lc, jnp.int32(TRASH))
        sub, off = (k * L) // 80, (k * L) % 80
        cidx[b * SUBS + sub][pl.ds(off, L)] = lc


def _edge_pass(vin, row, col, ridx, ctmp, cidx, rows, acc_sh,
               sem_i, sem_g, sem_s, nodebase, ebase):
    def scat_desc(b, sub):
        return pltpu.make_async_copy(
            rows[b].at[pl.ds(sub * 80, 80)],
            acc_sh.at[cidx[b * SUBS + sub]],
            sem_s.at[b])

    def super_body(g, _):
        e0 = ebase + g * (NBUF * EPC)
        descs = []
        for b in range(NBUF):
            @pl.when(g > 0)
            def _wait_prev(b=b):
                for sub in range(SUBS):
                    scat_desc(b, sub).wait()
            di = pltpu.async_copy(
                row.at[pl.ds(e0 + b * EPC, EPC)], ridx[b], sem_i.at[b])
            dc = pltpu.async_copy(
                col.at[pl.ds(e0 + b * EPC, EPC)], ctmp[b], sem_i.at[b])
            descs.append((di, dc))
        gd = []
        for b in range(NBUF):
            descs[b][0].wait()
            descs[b][1].wait()
            _remap_store(ctmp, cidx, b, nodebase)
            gd.append(pltpu.async_copy(
                vin.at[ridx[b]], rows[b], sem_g.at[b]))
        for b in range(NBUF):
            gd[b].wait()
            for sub in range(SUBS):
                pltpu.async_copy(
                    rows[b].at[pl.ds(sub * 80, 80)],
                    acc_sh.at[cidx[b * SUBS + sub]],
                    sem_s.at[b], add=True)
        return _

    lax.fori_loop(0, NCH // NBUF, super_body, None)
    for b in range(NBUF):
        for sub in range(SUBS):
            scat_desc(b, sub).wait()


def _make_prop(ch, nhalf):
    oshape = jax.ShapeDtypeStruct((NPAD, ch), jnp.float32)
    outs = tuple([oshape] * (2 * nhalf))

    scratch = [
        pltpu.VMEM((ZPT, ch), jnp.float32),
    ] + [pltpu.VMEM((EPC,), jnp.int32) for _ in range(NBUF)] + [
        pltpu.VMEM((EPC,), jnp.int32) for _ in range(NBUF)] + [
        pltpu.VMEM((80,), jnp.int32) for _ in range(NBUF * SUBS)] + [
        pltpu.VMEM((EPC, ch), jnp.float32) for _ in range(NBUF)] + [
        pltpu.VMEM((160, ch), jnp.float32),
        pltpu.VMEM((160, ch), jnp.float32),
        pltpu.VMEM((PT,), jnp.float32),
        pltpu.VMEM_SHARED((ACC_ROWS, ch), jnp.float32),
        pltpu.SemaphoreType.DMA((NBUF,)),
        pltpu.SemaphoreType.DMA((NBUF,)),
        pltpu.SemaphoreType.DMA((NBUF,)),
    ]

    def body(*refs):
        vins = refs[:nhalf]
        row, col, dis = refs[nhalf:nhalf + 3]
        prevs = refs[nhalf + 3:2 * nhalf + 3]
        tks = refs[2 * nhalf + 3:3 * nhalf + 3]
        vks = refs[3 * nhalf + 3:4 * nhalf + 3]
        refs = refs[4 * nhalf + 3:]
        zb = refs[0]
        ctmp = refs[1:1 + NBUF]
        ridx = refs[1 + NBUF:1 + 2 * NBUF]
        cidx = refs[1 + 2 * NBUF:1 + 2 * NBUF + NBUF * SUBS]
        rows = refs[1 + 2 * NBUF + NBUF * SUBS:1 + 3 * NBUF + NBUF * SUBS]
        (abuf, pbuf, disb, acc_sh, sem_i, sem_g,
         sem_s) = refs[1 + 3 * NBUF + NBUF * SUBS:]

        cc = lax.axis_index("c")
        s = lax.axis_index("s")
        nodebase = cc * HALFP
        ebase = s * EPT

        pltpu.sync_copy(dis.at[pl.ds(nodebase + s * PT, PT)], disb)

        for half in range(nhalf):
            _zero_acc(zb, acc_sh, s, ch)
            plsc.subcore_barrier()
            _edge_pass(vins[half], row, col, ridx, ctmp, cidx, rows,
                       acc_sh, sem_i, sem_g, sem_s, nodebase, ebase)
            plsc.subcore_barrier()

            for it in range(2):
                r0 = s * PT + it * 160
                pltpu.sync_copy(acc_sh.at[pl.ds(r0, 160)], abuf)
                pltpu.sync_copy(prevs[half].at[pl.ds(nodebase + r0, 160)],
                                pbuf)

                def comb(n, _, it=it):
                    splat = plsc.load_gather(
                        disb, [jnp.zeros((L,), jnp.int32) + (it * 160 + n)])
                    for jb in range(ch // L):
                        sl = pl.ds(jb * L, L)
                        t = -2.0 * splat * abuf[n, sl] - pbuf[n, sl]
                        abuf[n, sl] = t
                        pbuf[n, sl] = splat * t
                    return _
                lax.fori_loop(0, 160, comb, None)
                pltpu.sync_copy(abuf, tks[half].at[pl.ds(nodebase + r0, 160)])
                pltpu.sync_copy(pbuf, vks[half].at[pl.ds(nodebase + r0, 160)])
            if half + 1 < nhalf:
                plsc.subcore_barrier()

    mesh = plsc.VectorSubcoreMesh(**_MESH)
    return pl.kernel(body, out_type=outs, mesh=mesh, scratch_types=scratch,
                     compiler_params=_SC_PARAMS)


def _make_deg():
    DEPC = 80
    scratch = [
        pltpu.VMEM((ZPT, L), jnp.float32),
        pltpu.VMEM((80, L), jnp.float32),
    ] + [pltpu.VMEM((DEPC,), jnp.int32) for _ in range(NBUF)] + [
        pltpu.VMEM((80,), jnp.int32) for _ in range(NBUF)] + [
        pltpu.VMEM((PT, L), jnp.float32),
        pltpu.VMEM((PT,), jnp.float32),
        pltpu.VMEM((160, IN // 2), jnp.float32),
        pltpu.VMEM_SHARED((ACC_ROWS, L), jnp.float32),
        pltpu.SemaphoreType.DMA((NBUF,)),
        pltpu.SemaphoreType.DMA((NBUF,)),
    ]

    def body(row, xL, xR, dis_out, v0L, v0R, zb, ones, *refs):
        rtmp = refs[:NBUF]
        sidx = refs[NBUF:2 * NBUF]
        accb, disb, xb, acc_sh, sem_i, sem_s = refs[2 * NBUF:]
        cc = lax.axis_index("c")
        s = lax.axis_index("s")
        nodebase = cc * HALFP
        ebase = s * EPT

        _zero_acc(zb, acc_sh, s, L)

        def onesrow(r, _):
            ones[r, pl.ds(0, L)] = jnp.ones((L,), jnp.float32)
            return _
        lax.fori_loop(0, 80, onesrow, None)
        plsc.subcore_barrier()

        def scat_desc(b):
            return pltpu.make_async_copy(
                ones, acc_sh.at[sidx[b]], sem_s.at[b])

        def super_body(g, _):
            e0 = ebase + g * (NBUF * DEPC)
            descs = []
            for b in range(NBUF):
                @pl.when(g > 0)
                def _wait_prev(b=b):
                    scat_desc(b).wait()
                descs.append(pltpu.async_copy(
                    row.at[pl.ds(e0 + b * DEPC, DEPC)], rtmp[b],
                    sem_i.at[b]))
            for b in range(NBUF):
                descs[b].wait()
                for k in range(DEPC // L):
                    v = rtmp[b][pl.ds(k * L, L)]
                    lc = v - nodebase
                    okm = (lc >= 0) & (lc < HALFP)
                    lc = jnp.where(okm, lc, jnp.int32(TRASH))
                    sidx[b][pl.ds(k * L, L)] = lc
                pltpu.async_copy(ones, acc_sh.at[sidx[b]],
                                 sem_s.at[b], add=True)
            return _

        lax.fori_loop(0, (EPT // DEPC) // NBUF, super_body, None)
        for b in range(NBUF):
            scat_desc(b).wait()
        plsc.subcore_barrier()

        pltpu.sync_copy(acc_sh.at[pl.ds(s * PT, PT)], accb)
        lanes = lax.iota(jnp.int32, L)
        zeros = jnp.zeros((L,), jnp.int32)

        def disrow(v, _):
            d = plsc.load_gather(accb, [v * L + lanes, zeros])
            r = _fast_rsqrt(jnp.maximum(d, 1e-12))
            disb[pl.ds(v * L, L)] = jnp.where(d >= 0.5, r, 0.0)
            return _
        lax.fori_loop(0, PT // L, disrow, None)
        pltpu.sync_copy(disb, dis_out.at[pl.ds(nodebase + s * PT, PT)])

        for half, (xh, vh) in enumerate(((xL, v0L), (xR, v0R))):
            for it in range(2):
                r0 = nodebase + s * PT + it * 160
                pltpu.sync_copy(xh.at[pl.ds(r0, 160)], xb)

                def scale(n, _, it=it):
                    splat = plsc.load_gather(
                        disb, [jnp.zeros((L,), jnp.int32) + (it * 160 + n)])
                    for jb in range((IN // 2) // L):
                        sl = pl.ds(jb * L, L)
                        xb[n, sl] = xb[n, sl] * (0.5 * splat)
                    return _
                lax.fori_loop(0, 160, scale, None)
                pltpu.sync_copy(xb, vh.at[pl.ds(r0, 160)])

    mesh = plsc.VectorSubcoreMesh(**_MESH)
    h64 = jax.ShapeDtypeStruct((NPAD, IN // 2), jnp.float32)
    return pl.kernel(
        body,
        out_type=(jax.ShapeDtypeStruct((NPAD,), jnp.float32), h64, h64),
        mesh=mesh, scratch_types=scratch, compiler_params=_SC_PARAMS)


def _make_final():
    C = 16
    scratch = [
        pltpu.VMEM((ZPT, C), jnp.float32),
    ] + [pltpu.VMEM((EPC,), jnp.int32) for _ in range(NBUF)] + [
        pltpu.VMEM((EPC,), jnp.int32) for _ in range(NBUF)] + [
        pltpu.VMEM((80,), jnp.int32) for _ in range(NBUF * SUBS)] + [
        pltpu.VMEM((EPC, C), jnp.float32) for _ in range(NBUF)] + [
        pltpu.VMEM((PT, C), jnp.float32),
        pltpu.VMEM((PT, C), jnp.float32),
        pltpu.VMEM((PT, C), jnp.float32),
        pltpu.VMEM((PT, C), jnp.float32),
        pltpu.VMEM((PT // 4, 16), jnp.float32),
        pltpu.VMEM((PT,), jnp.float32),
        pltpu.VMEM((L,), jnp.float32),
        pltpu.VMEM_SHARED((ACC_ROWS, C), jnp.float32),
        pltpu.SemaphoreType.DMA((NBUF,)),
        pltpu.SemaphoreType.DMA((NBUF,)),
        pltpu.SemaphoreType.DMA((NBUF,)),
    ]

    def body(vin, row, col, dis, g, p1, m, b2t, y, *refs):
        zb = refs[0]
        ctmp = refs[1:1 + NBUF]
        ridx = refs[1 + NBUF:1 + 2 * NBUF]
        cidx = refs[1 + 2 * NBUF:1 + 2 * NBUF + NBUF * SUBS]
        rows = refs[1 + 2 * NBUF + NBUF * SUBS:1 + 3 * NBUF + NBUF * SUBS]
        (accb, gb, p1b, mb, yb, disb, b2b, acc_sh,
         sem_i, sem_g, sem_s) = refs[1 + 3 * NBUF + NBUF * SUBS:]

        cc = lax.axis_index("c")
        s = lax.axis_index("s")
        nodebase = cc * HALFP
        ebase = s * EPT

        _zero_acc(zb, acc_sh, s, C)
        plsc.subcore_barrier()
        _edge_pass(vin, row, col, ridx, ctmp, cidx, rows, acc_sh,
                   sem_i, sem_g, sem_s, nodebase, ebase)
        plsc.subcore_barrier()

        n0 = nodebase + s * PT
        pltpu.sync_copy(acc_sh.at[pl.ds(s * PT, PT)], accb)
        pltpu.sync_copy(g.at[pl.ds(n0, PT)], gb)
        pltpu.sync_copy(p1.at[pl.ds(n0, PT)], p1b)
        pltpu.sync_copy(m.at[pl.ds(n0, PT)], mb)
        pltpu.sync_copy(dis.at[pl.ds(n0, PT)], disb)
        pltpu.sync_copy(b2t, b2b)

        lanes = lax.iota(jnp.int32, L)
        nd = lax.shift_right_logical(lanes, 2)
        jm = lanes & 3
        b2v = b2b[pl.ds(0, L)]

        def comb(q, _):
            nidx = q * 4 + nd
            disv = plsc.load_gather(disb, [nidx])
            accv = plsc.load_gather(accb, [nidx, 12 + jm])
            gv = plsc.load_gather(gb, [nidx, jm])
            p4 = plsc.load_gather(p1b, [nidx, 4 + jm])
            p12 = plsc.load_gather(p1b, [nidx, 12 + jm])
            m8 = plsc.load_gather(mb, [nidx, 8 + jm])
            yv = gv + p4 + m8 - 2.0 * disv * accv - p12 + b2v
            yb[q, pl.ds(0, L)] = jnp.maximum(yv, 0.0)
            return _
        lax.fori_loop(0, PT // 4, comb, None)
        pltpu.sync_copy(yb, y.at[pl.ds(n0 // 4, PT // 4)])

    mesh = plsc.VectorSubcoreMesh(**_MESH)
    return pl.kernel(
        body,
        out_type=jax.ShapeDtypeStruct((NPAD // 4, 16), jnp.float32),
        mesh=mesh, scratch_types=scratch, compiler_params=_SC_PARAMS)


def _tc_matmul(xp, t1l, t1r, t2l, t2r, t3l, t3r, w1cat, b1r, w2cat, dis2d):
    R = 512
    H = IN // 2
    grid = (NPAD // R,)

    def body(x_r, a_r, b_r, c_r, d_r, e_r, f_r, w1_r, b1_r, w2_r, s_r,
             g_r, vg_r):
        tcat = jnp.concatenate(
            [x_r[...], a_r[...], b_r[...], c_r[...], d_r[...], e_r[...],
             f_r[...]], axis=1)
        h = jnp.dot(tcat, w1_r[...], preferred_element_type=jnp.float32)
        h = jnp.maximum(h + b1_r[...], 0.0)
        g = jnp.dot(h, w2_r[...], preferred_element_type=jnp.float32)
        g_r[...] = g
        vg_r[...] = (0.5 * s_r[...]) * g

    full = lambda shape: pl.BlockSpec(shape, lambda i: (0, 0))
    rows = lambda w: pl.BlockSpec((R, w), lambda i: (i, 0))
    return pl.pallas_call(
        body,
        grid=grid,
        in_specs=[rows(IN), rows(H), rows(H), rows(H), rows(H), rows(H),
                  rows(H),
                  full((K * IN, HID)), full((1, HID)), full((HID, K * OUT)),
                  rows(1)],
        out_specs=[rows(K * OUT), rows(K * OUT)],
        out_shape=[jax.ShapeDtypeStruct((NPAD, K * OUT), jnp.float32),
                   jax.ShapeDtypeStruct((NPAD, K * OUT), jnp.float32)],
    )(xp, t1l, t1r, t2l, t2r, t3l, t3r, w1cat, b1r, w2cat, dis2d)


_deg_call = _make_deg()
_prop128 = _make_prop(IN // 2, nhalf=2)
_prop16 = _make_prop(K * OUT, nhalf=1)
_final_call = _make_final()


def kernel(x, edge_index, W1, b1, W2, b2):
    row = edge_index[0].astype(jnp.int32)
    col = edge_index[1].astype(jnp.int32)
    xp = jnp.zeros((NPAD, IN), jnp.float32).at[:N].set(x)
    xl, xr = xp[:, :IN // 2], xp[:, IN // 2:]
    w1cat = W1.reshape(K * IN, HID)
    w2cat = jnp.transpose(W2, (1, 0, 2)).reshape(HID, K * OUT)
    b1r = b1.reshape(1, HID)
    b2t = jnp.tile(b2, 4)
    z64 = jnp.zeros((NPAD, IN // 2), jnp.float32)
    z16 = jnp.zeros((NPAD, K * OUT), jnp.float32)

    dis, v0l, v0r = _deg_call(row, xl, xr)
    t1l, t1r, v1l, v1r = _prop128(v0l, v0r, row, col, dis, z64, z64)
    t2l, t2r, v2l, v2r = _prop128(v1l, v1r, row, col, dis, xl, xr)
    t3l, t3r, _u1, _u2 = _prop128(v2l, v2r, row, col, dis, t1l, t1r)
    g, vg = _tc_matmul(xp, t1l, t1r, t2l, t2r, t3l, t3r, w1cat, b1r, w2cat,
                       dis.reshape(NPAD, 1))
    p1, v1p = _prop16(vg, row, col, dis, z16)
    m, v2p = _prop16(v1p, row, col, dis, g)
    y = _final_call(v2p, row, col, dis, g, p1, m, b2t)
    return y.reshape(NPAD, OUT)[:N]
